# baseline (speedup 1.0000x reference)
import jax
import jax.numpy as jnp
from jax import lax
from jax.experimental import pallas as pl
from jax.experimental.pallas import tpu as pltpu
from jax.experimental.pallas import tpu_sc as plsc

B, T, F, D = 1024, 50, 26, 32
TF = T * F
L = 16
NC, NS = 2, 16
NW = NC * NS
ROWS = B // NW
CH = 160
NCH = TF // CH
TAIL = TF - NCH * CH
XPAD = 1312


def _bias_body(t_ref, f_ref, bv_ref, out_ref):
    t = t_ref[...]
    f = f_ref[...]
    bv = bv_ref[...]
    out_ref[...] = t[:, None, :] + f[None, :, :] + bv[0][None, None, :]


def _bcast(xv, idx):
    return lax.gather(
        xv, idx[:, None],
        lax.GatherDimensionNumbers(
            offset_dims=(), collapsed_slice_dims=(0,), start_index_map=(0,)),
        slice_sizes=(1,),
        mode=lax.GatherScatterMode.PROMISE_IN_BOUNDS,
    )


def _sc_body(x_ref, bias_ref, w_ref, out_ref,
             bias_v, buf0, buf1, tbuf, xslab, w_v, sem0, sem1, sem2):
    wid = lax.axis_index("s") * NC + lax.axis_index("c")
    base = wid * ROWS

    pltpu.sync_copy(bias_ref, bias_v)
    pltpu.sync_copy(w_ref, w_v)
    wlo = w_v[0:L]
    whi = w_v[L:D]
    bidx = [jnp.full((L,), j, jnp.int32) for j in range(L)]

    bufs = (buf0, buf1)
    sems = (sem0, sem1)

    @pl.loop(0, ROWS // 8)
    def _slab(s):
        b8 = base + s * 8
        pltpu.sync_copy(x_ref.at[pl.ds(b8, 8)], xslab)

        @pl.loop(0, 8)
        def _row(k):
            b = b8 + k

            @pl.loop(0, NCH, step=2)
            def _chunk(cc):
                for p in range(2):
                    c = (cc + p) * CH
                    buf = bufs[p]
                    sem = sems[p]

                    @pl.when((s > 0) | (k > 0) | (cc > 0))
                    def _():
                        pltpu.make_async_copy(
                            buf, out_ref.at[b, pl.ds(c, CH)], sem).wait()

                    @plsc.parallel_loop(0, CH, L, unroll=2)
                    def _tf(t0):
                        xv = xslab[k, pl.ds(c + t0, L)]
                        for j in range(L):
                            xs = _bcast(xv, bidx[j])
                            o = (c + t0 + j) * D
                            buf[t0 + j, 0:L] = xs * wlo + bias_v[pl.ds(o, L)]
                            buf[t0 + j, L:D] = xs * whi + bias_v[pl.ds(o + L, L)]

                    pltpu.async_copy(buf, out_ref.at[b, pl.ds(c, CH)], sem)

            @pl.when((s > 0) | (k > 0))
            def _():
                pltpu.make_async_copy(
                    tbuf, out_ref.at[b, pl.ds(NCH * CH, TAIL)], sem2).wait()

            for t0 in range(0, 32, L):
                xv = xslab[k, pl.ds(NCH * CH + t0, L)]
                for j in range(L):
                    t = t0 + j
                    if t >= TAIL:
                        break
                    xs = _bcast(xv, bidx[j])
                    o = (NCH * CH + t) * D
                    tbuf[t, 0:L] = xs * wlo + bias_v[pl.ds(o, L)]
                    tbuf[t, L:D] = xs * whi + bias_v[pl.ds(o + L, L)]

            pltpu.async_copy(tbuf, out_ref.at[b, pl.ds(NCH * CH, TAIL)], sem2)

    last = base + ROWS - 1
    pltpu.make_async_copy(buf0, out_ref.at[last, pl.ds(0, CH)], sem0).wait()
    pltpu.make_async_copy(buf1, out_ref.at[last, pl.ds(CH, CH)], sem1).wait()
    pltpu.make_async_copy(tbuf, out_ref.at[last, pl.ds(NCH * CH, TAIL)], sem2).wait()


def kernel(x, W_val, b_val, time_embed, feat_embed):
    bias3 = pl.pallas_call(
        _bias_body,
        out_shape=jax.ShapeDtypeStruct((T, F, D), jnp.float32),
    )(time_embed, feat_embed, b_val.reshape(1, D))
    bias1 = bias3.reshape(TF * D)

    x2 = jnp.pad(x.reshape(B, TF), ((0, 0), (0, XPAD - TF)))

    sc = pl.kernel(
        _sc_body,
        out_type=jax.ShapeDtypeStruct((B, TF, D), jnp.float32),
        mesh=plsc.VectorSubcoreMesh(core_axis_name="c", subcore_axis_name="s"),
        scratch_types=[
            pltpu.VMEM((TF * D,), jnp.float32),
            pltpu.VMEM((CH, D), jnp.float32),
            pltpu.VMEM((CH, D), jnp.float32),
            pltpu.VMEM((TAIL, D), jnp.float32),
            pltpu.VMEM((8, XPAD), jnp.float32),
            pltpu.VMEM((D,), jnp.float32),
            pltpu.SemaphoreType.DMA,
            pltpu.SemaphoreType.DMA,
            pltpu.SemaphoreType.DMA,
        ],
    )
    return sc(x2, bias1, W_val.reshape(D))


# speedup vs baseline: 1.0535x; 1.0535x over previous
import jax
import jax.numpy as jnp
from jax.experimental import pallas as pl
from jax.experimental.pallas import tpu as pltpu

B, T, F, D = 1024, 50, 26, 32
TF = T * F
BBLK = 32


def _bias_body(t_ref, f_ref, bv_ref, out_ref):
    t = t_ref[...]
    f = f_ref[...]
    bv = bv_ref[...]
    out_ref[...] = t[:, None, :] + f[None, :, :] + bv[0][None, None, :]


def _main_body(x_ref, w_ref, bias_ref, out_ref):
    x = x_ref[...]
    w = w_ref[0]
    bias = bias_ref[...]
    out_ref[...] = x[:, :, None] * w[None, None, :] + bias[None, :, :]


def kernel(x, W_val, b_val, time_embed, feat_embed):
    bias3 = pl.pallas_call(
        _bias_body,
        out_shape=jax.ShapeDtypeStruct((T, F, D), jnp.float32),
    )(time_embed, feat_embed, b_val.reshape(1, D))
    bias2 = bias3.reshape(TF, D)

    x2 = x.reshape(B, TF)

    out = pl.pallas_call(
        _main_body,
        grid=(B // BBLK,),
        in_specs=[
            pl.BlockSpec((BBLK, TF), lambda i: (i, 0)),
            pl.BlockSpec((1, D), lambda i: (0, 0)),
            pl.BlockSpec((TF, D), lambda i: (0, 0)),
        ],
        out_specs=pl.BlockSpec((BBLK, TF, D), lambda i: (i, 0, 0)),
        out_shape=jax.ShapeDtypeStruct((B, TF, D), jnp.float32),
    )(x2, W_val, bias2)

    return out
